# baseline (speedup 1.0000x reference)
import functools

import jax
import jax.numpy as jnp
from jax.experimental import pallas as pl
from jax.experimental.pallas import tpu as pltpu

B = 32
D = 1024
V = 100000
TOP_P = 0.9

BN = 4096
VPAD = ((V + BN - 1) // BN) * BN


def _matmul_body(x_ref, w_ref, b_ref, o_ref):
    o_ref[...] = (
        jax.lax.dot_general(
            x_ref[...], w_ref[...], (((1,), (0,)), ((), ())),
            preferred_element_type=jnp.float32,
        )
        + b_ref[...]
    )


def _logits(x, W, b):
    grid = VPAD // BN
    return pl.pallas_call(
        _matmul_body,
        grid=(grid,),
        in_specs=[
            pl.BlockSpec((B, D), lambda i: (0, 0)),
            pl.BlockSpec((D, BN), lambda i: (0, i)),
            pl.BlockSpec((BN,), lambda i: (i,)),
        ],
        out_specs=pl.BlockSpec((B, BN), lambda i: (0, i)),
        out_shape=jax.ShapeDtypeStruct((B, VPAD), jnp.float32),
    )(x, W, b)


def _softmax_body(l_ref, p_ref):
    x = l_ref[...]
    valid = jax.lax.broadcasted_iota(jnp.int32, (B, VPAD), 1) < V
    x = jnp.where(valid, x, -jnp.inf)
    xmax = jnp.max(x, axis=1, keepdims=True)
    u = jnp.exp(x - xmax)
    s = jnp.sum(u, axis=1, keepdims=True)
    p_ref[...] = u / s


def _probs(logits):
    return pl.pallas_call(
        _softmax_body,
        out_shape=jax.ShapeDtypeStruct((B, VPAD), jnp.float32),
    )(logits)


def kernel(batch, W, b):
    x = batch[:, -1, :]
    bpad = jnp.pad(b, (0, VPAD - V))
    logits = _logits(x, W, bpad)
    p = _probs(logits)[:, :V]

    keys = jax.random.split(jax.random.key(42), B)

    def sample_top_p(p_row, key):
        order = jnp.argsort(-p_row)
        sorted_probs = p_row[order]
        cumulative = jnp.cumsum(sorted_probs)
        p_eff = jnp.maximum(cumulative[0], TOP_P)
        mask = cumulative <= p_eff
        masked = jnp.where(mask, sorted_probs, 0.0)
        masked = masked / jnp.sum(masked)
        logp = jnp.where(mask, jnp.log(jnp.maximum(masked, 1e-38)), -jnp.inf)
        sampled_index = jax.random.categorical(key, logp)
        return order[sampled_index].astype(jnp.int32)

    return jax.vmap(sample_top_p)(p, keys)
